# Initial kernel scaffold; baseline (speedup 1.0000x reference)
import jax
import jax.numpy as jnp
from jax.experimental import pallas as pl


def kernel(input, prompt_table, normal_table):
    raise NotImplementedError("write your pallas kernel here")


# speedup vs baseline: 3.7286x; 3.7286x over previous
import functools

import jax
import jax.numpy as jnp
from jax import lax
from jax.experimental import pallas as pl
from jax.experimental.pallas import tpu as pltpu
from jax.experimental.pallas import tpu_sc as plsc

PROMPT_LEN = 100
EMBED = 64


def kernel(input, prompt_table, normal_table):
    B, S = input.shape
    assert S == 2 * PROMPT_LEN
    info = plsc.get_sparse_core_info()
    num_workers = info.num_cores * info.num_subcores
    rows_per_w = B // num_workers

    inp3 = input.reshape(B, 2, PROMPT_LEN)
    mesh = plsc.VectorSubcoreMesh(core_axis_name="c", subcore_axis_name="s")

    @functools.partial(
        pl.kernel,
        out_type=jax.ShapeDtypeStruct((B, S, EMBED), jnp.float32),
        mesh=mesh,
        scratch_types=[
            pltpu.VMEM((2, PROMPT_LEN), jnp.int32),
            pltpu.VMEM((S, EMBED), jnp.float32),
            pltpu.SemaphoreType.DMA,
            pltpu.SemaphoreType.DMA,
        ],
        compiler_params=pltpu.CompilerParams(use_tc_tiling_on_sc=False),
    )
    def emb(inp_hbm, ptab_hbm, ntab_hbm, out_hbm, idx_v, rows_v, sem1, sem2):
        wid = lax.axis_index("s") * info.num_cores + lax.axis_index("c")
        base = wid * rows_per_w

        def body(i, carry):
            b = base + i
            pltpu.sync_copy(inp_hbm.at[b], idx_v)
            cp1 = pltpu.async_copy(
                ptab_hbm.at[idx_v.at[0]], rows_v.at[pl.ds(0, PROMPT_LEN)], sem1
            )
            cp2 = pltpu.async_copy(
                ntab_hbm.at[idx_v.at[1]], rows_v.at[pl.ds(PROMPT_LEN, PROMPT_LEN)], sem2
            )
            cp1.wait()
            cp2.wait()
            pltpu.sync_copy(rows_v, out_hbm.at[b])
            return carry

        lax.fori_loop(0, rows_per_w, body, 0)

    return emb(inp3, prompt_table, normal_table)


# speedup vs baseline: 3.8871x; 1.0425x over previous
import functools

import jax
import jax.numpy as jnp
from jax import lax
from jax.experimental import pallas as pl
from jax.experimental.pallas import tpu as pltpu
from jax.experimental.pallas import tpu_sc as plsc

PROMPT_LEN = 100
EMBED = 64
RPC = 2
NBUF = 2


def kernel(input, prompt_table, normal_table):
    B, S = input.shape
    assert S == 2 * PROMPT_LEN
    info = plsc.get_sparse_core_info()
    num_workers = info.num_cores * info.num_subcores
    rows_per_w = B // num_workers
    nchunks = rows_per_w // RPC

    inp3 = input.reshape(B, 2, PROMPT_LEN)
    mesh = plsc.VectorSubcoreMesh(core_axis_name="c", subcore_axis_name="s")

    @functools.partial(
        pl.kernel,
        out_type=jax.ShapeDtypeStruct((B * S, EMBED), jnp.float32),
        mesh=mesh,
        scratch_types=[
            pltpu.VMEM((rows_per_w, 2, PROMPT_LEN), jnp.int32),
            [pltpu.VMEM((RPC * S, EMBED), jnp.float32) for _ in range(NBUF)],
            [pltpu.SemaphoreType.DMA for _ in range(NBUF)],
            [pltpu.SemaphoreType.DMA for _ in range(NBUF)],
        ],
        compiler_params=pltpu.CompilerParams(use_tc_tiling_on_sc=False),
    )
    def emb(inp_hbm, ptab_hbm, ntab_hbm, out_hbm, idx_v, rows_v, gsems, ssems):
        wid = lax.axis_index("s") * info.num_cores + lax.axis_index("c")
        row0 = wid * rows_per_w
        out0 = row0 * S

        pltpu.sync_copy(inp_hbm.at[pl.ds(row0, rows_per_w)], idx_v)

        def fire_gathers(c, b):
            for r in range(RPC):
                row = c * RPC + r
                pltpu.async_copy(
                    ptab_hbm.at[idx_v.at[row, 0]],
                    rows_v[b].at[pl.ds(r * S, PROMPT_LEN)],
                    gsems[b],
                )
                pltpu.async_copy(
                    ntab_hbm.at[idx_v.at[row, 1]],
                    rows_v[b].at[pl.ds(r * S + PROMPT_LEN, PROMPT_LEN)],
                    gsems[b],
                )

        def wait_gathers(b):
            for r in range(RPC):
                pltpu.make_async_copy(
                    ptab_hbm.at[idx_v.at[0, 0]],
                    rows_v[b].at[pl.ds(r * S, PROMPT_LEN)],
                    gsems[b],
                ).wait()
                pltpu.make_async_copy(
                    ntab_hbm.at[idx_v.at[0, 1]],
                    rows_v[b].at[pl.ds(r * S + PROMPT_LEN, PROMPT_LEN)],
                    gsems[b],
                ).wait()

        def fire_store(c, b):
            pltpu.async_copy(
                rows_v[b], out_hbm.at[pl.ds(out0 + c * (RPC * S), RPC * S)], ssems[b]
            )

        def wait_store(b):
            pltpu.make_async_copy(
                rows_v[b], out_hbm.at[pl.ds(out0, RPC * S)], ssems[b]
            ).wait()

        for b in range(NBUF):
            fire_gathers(b, b)

        def body(g, carry):
            for b in range(NBUF):
                c = g * NBUF + b
                wait_gathers(b)
                fire_store(c, b)
            for b in range(NBUF):
                c = g * NBUF + b
                wait_store(b)
                fire_gathers(c + NBUF, b)
            return carry

        lax.fori_loop(0, nchunks // NBUF - 1, body, 0)

        for b in range(NBUF):
            c = nchunks - NBUF + b
            wait_gathers(b)
            fire_store(c, b)
        for b in range(NBUF):
            wait_store(b)

    out = emb(inp3, prompt_table, normal_table)
    return out.reshape(B, S, EMBED)
